# Initial kernel scaffold; baseline (speedup 1.0000x reference)
import jax
import jax.numpy as jnp
from jax.experimental import pallas as pl


def kernel(predictions, targets):
    raise NotImplementedError("write your pallas kernel here")


# speedup vs baseline: 26.9494x; 26.9494x over previous
import functools

import jax
import jax.numpy as jnp
from jax.experimental import pallas as pl
from jax.experimental.pallas import tpu as pltpu

_K_PERCENT = 0.2
_BATCH = 64
_N = 32768
_BLOCK_ROWS = 8


def _loss_kernel(pred_ref, targ_ref, out_ref, *, k, n, num_blocks):
    pid = pl.program_id(0)

    t = targ_ref[...]
    p = pred_ref[...]

    u = pltpu.bitcast(t, jnp.uint32)
    sign = (u >> 31).astype(jnp.uint32)
    flip = jnp.where(sign == 1, jnp.uint32(0xFFFFFFFF), jnp.uint32(0x80000000))
    key = u ^ flip

    kk = jnp.int32(k)

    def bit_step(i, carry):
        p_top, p_bot = carry
        bit = jnp.uint32(1) << (jnp.uint32(31) - i.astype(jnp.uint32))
        cand_t = p_top | bit
        cand_b = p_bot | bit
        cnt_t = jnp.sum((key >= cand_t).astype(jnp.int32), axis=1, keepdims=True)
        cnt_b = jnp.sum((key <= ~cand_b).astype(jnp.int32), axis=1, keepdims=True)
        p_top = jnp.where(cnt_t >= kk, cand_t, p_top)
        p_bot = jnp.where(cnt_b >= kk, cand_b, p_bot)
        return p_top, p_bot

    rows = t.shape[0]
    z = jnp.zeros((rows, 1), jnp.uint32)
    t_top, t_bot_inv = jax.lax.fori_loop(0, 32, bit_step, (z, z))
    t_bot = ~t_bot_inv

    gt_top = key > t_top
    eq_top = key == t_top
    lt_bot = key < t_bot
    eq_bot = key == t_bot

    zf = jnp.float32(0.0)
    sum_gt = jnp.sum(jnp.where(gt_top, p, zf), axis=1, keepdims=True)
    sum_eqt = jnp.sum(jnp.where(eq_top, p, zf), axis=1, keepdims=True)
    cnt_gt = jnp.sum(gt_top.astype(jnp.int32), axis=1, keepdims=True)
    cnt_eqt = jnp.sum(eq_top.astype(jnp.int32), axis=1, keepdims=True)

    sum_lt = jnp.sum(jnp.where(lt_bot, p, zf), axis=1, keepdims=True)
    sum_eqb = jnp.sum(jnp.where(eq_bot, p, zf), axis=1, keepdims=True)
    cnt_lt = jnp.sum(lt_bot.astype(jnp.int32), axis=1, keepdims=True)
    cnt_eqb = jnp.sum(eq_bot.astype(jnp.int32), axis=1, keepdims=True)

    kf = jnp.float32(k)
    top_sum = sum_gt + (kf - cnt_gt.astype(jnp.float32)) * sum_eqt / cnt_eqt.astype(jnp.float32)
    bot_sum = sum_lt + (kf - cnt_lt.astype(jnp.float32)) * sum_eqb / cnt_eqb.astype(jnp.float32)
    margin = jnp.maximum(1.0 - (top_sum - bot_sum) / kf, 0.0)

    partial = jnp.sum(margin) / jnp.float32(_BATCH)

    @pl.when(pid == 0)
    def _init():
        out_ref[0, 0] = partial

    @pl.when(pid != 0)
    def _acc():
        out_ref[0, 0] += partial


def kernel(predictions, targets):
    n = targets.shape[1]
    k = max(1, int(n * _K_PERCENT))
    num_blocks = targets.shape[0] // _BLOCK_ROWS
    out = pl.pallas_call(
        functools.partial(_loss_kernel, k=k, n=n, num_blocks=num_blocks),
        grid=(num_blocks,),
        in_specs=[
            pl.BlockSpec((_BLOCK_ROWS, n), lambda i: (i, 0)),
            pl.BlockSpec((_BLOCK_ROWS, n), lambda i: (i, 0)),
        ],
        out_specs=pl.BlockSpec(memory_space=pltpu.SMEM),
        out_shape=jax.ShapeDtypeStruct((1, 1), jnp.float32),
        compiler_params=pltpu.CompilerParams(
            dimension_semantics=("arbitrary",),
        ),
    )(predictions, targets)
    return out[0, 0]
